# baseline (speedup 1.0000x reference)
import functools

import jax
import jax.numpy as jnp
import numpy as np
from jax import lax
from jax.experimental import pallas as pl
from jax.experimental.pallas import tpu as pltpu
from jax.experimental.pallas import tpu_sc as plsc

ROWS = 128
N = 100000
POWER = np.float32(0.1)

NC, NS = 2, 16
NW = NC * NS
CR = 200
CHUNKS = N // CR
CWORDS = CR * ROWS
FULLW = 20
NT_HI = 16
NT_LO = 15

_PERM = np.array([
    121, 35, 45, 99, 31, 112, 85, 63, 117, 114, 82, 65, 7, 4, 101, 102,
    78, 29, 108, 83, 44, 16, 58, 123, 37, 111, 19, 61, 2, 34, 5, 90,
    110, 72, 30, 42, 3, 70, 67, 39, 56, 69, 80, 22, 6, 118, 54, 77,
    18, 10, 11, 53, 94, 32, 15, 49, 50, 20, 43, 92, 8, 24, 81, 96,
    106, 9, 40, 71, 93, 59, 75, 97, 66, 25, 73, 13, 52, 88, 62, 87,
    76, 60, 47, 33, 79, 14, 17, 38, 86, 23, 105, 0, 41, 64, 21, 124,
    116, 26, 57, 89, 126, 125, 1, 115, 28, 113, 48, 36, 119, 120, 122, 100,
    91, 55, 103, 51, 127, 98, 107, 27, 74, 12, 109, 84, 68, 104, 95, 46,
], dtype=np.int32)
_PMATT = np.zeros((ROWS, ROWS), dtype=np.float32)
_PMATT[_PERM, np.arange(ROWS)] = 1.0


_RB = 5000
_RG = N // _RB


def _corr_body(xt_ref, pmat_ref, out_ref, acc_ref):
    j = pl.program_id(0)

    @pl.when(j == 0)
    def _():
        acc_ref[...] = jnp.zeros_like(acc_ref)

    acc_ref[...] += jnp.sum(xt_ref[...], axis=0, keepdims=True)

    @pl.when(j == _RG - 1)
    def _():
        permuted = jnp.dot(acc_ref[...], pmat_ref[...],
                           preferred_element_type=jnp.float32)
        out_ref[...] = jnp.broadcast_to(permuted * np.float32(POWER / N),
                                        (8, ROWS))


def _corrections(xt):
    return pl.pallas_call(
        _corr_body,
        grid=(_RG,),
        in_specs=[pl.BlockSpec((_RB, ROWS), lambda j: (j, 0)),
                  pl.BlockSpec((ROWS, ROWS), lambda j: (0, 0))],
        out_specs=pl.BlockSpec((8, ROWS), lambda j: (0, 0)),
        out_shape=jax.ShapeDtypeStruct((8, ROWS), jnp.float32),
        scratch_shapes=[pltpu.VMEM((1, ROWS), jnp.float32)],
    )(xt, jnp.asarray(_PMATT))


def _sc_body(xt_hbm, corr_hbm, perm_hbm, out_hbm,
             permv, corr_v, blk0, blk1, ob0, ob1, si0, si1, so0, so1):
    c = lax.axis_index("c")
    s = lax.axis_index("s")
    w = s * NC + c
    pltpu.sync_copy(perm_hbm, permv)
    pltpu.sync_copy(corr_hbm, corr_v)

    pidx = [permv[pl.ds(16 * g, 16)] for g in range(8)]
    cks = [corr_v[0, pl.ds(16 * g, 16)] for g in range(8)]

    blk = (blk0, blk1)
    ob = (ob0, ob1)
    sis = (si0, si1)
    sos = (so0, so1)
    nt = jnp.where(w < FULLW, NT_HI, NT_LO)

    def _src(t):
        return xt_hbm.at[pl.ds((w + NW * t) * CWORDS, CWORDS)]

    def _dst(t):
        return out_hbm.at[pl.ds((w + NW * t) * CWORDS, CWORDS)]

    def _chunk(t, b):
        pltpu.make_async_copy(_src(t), blk[b], sis[b]).wait()

        @pl.when(t >= 1)
        def _():
            pltpu.make_async_copy(ob[1 - b], _dst(t - 1), sos[1 - b]).wait()

        @pl.when(t <= nt - 2)
        def _():
            pltpu.async_copy(_src(t + 1), blk[1 - b], sis[1 - b])

        def _cmb(r, _, _b=b):
            base = r * ROWS
            for g in range(8):
                o = base + 16 * g
                x = blk[_b][pl.ds(o, 16)]
                nz = plsc.load_gather(blk[_b], [base + pidx[g]])
                ob[_b][pl.ds(o, 16)] = x + POWER * nz - cks[g]
            return 0

        lax.fori_loop(0, CR, _cmb, 0)
        pltpu.async_copy(ob[b], _dst(t), sos[b])

    def _body(t, _):
        pl.when(t % 2 == 0)(lambda: _chunk(t, 0))
        pl.when(t % 2 == 1)(lambda: _chunk(t, 1))
        return 0

    pltpu.async_copy(_src(0), blk[0], sis[0])
    lax.fori_loop(0, nt, _body, 0)

    @pl.when(w < FULLW)
    def _():
        pltpu.make_async_copy(ob[(NT_HI - 1) % 2], _dst(NT_HI - 1),
                              sos[(NT_HI - 1) % 2]).wait()

    @pl.when(w >= FULLW)
    def _():
        pltpu.make_async_copy(ob[(NT_LO - 1) % 2], _dst(NT_LO - 1),
                              sos[(NT_LO - 1) % 2]).wait()


def _sc_combine(xt_flat, corrw, perm):
    fn = pl.kernel(
        _sc_body,
        out_type=jax.ShapeDtypeStruct((N * ROWS,), jnp.float32),
        mesh=plsc.VectorSubcoreMesh(core_axis_name="c", subcore_axis_name="s"),
        scratch_types=[
            pltpu.VMEM((ROWS,), jnp.int32),
            pltpu.VMEM((8, ROWS), jnp.float32),
            pltpu.VMEM((CWORDS,), jnp.float32),
            pltpu.VMEM((CWORDS,), jnp.float32),
            pltpu.VMEM((CWORDS,), jnp.float32),
            pltpu.VMEM((CWORDS,), jnp.float32),
            pltpu.SemaphoreType.DMA,
            pltpu.SemaphoreType.DMA,
            pltpu.SemaphoreType.DMA,
            pltpu.SemaphoreType.DMA,
        ],
        compiler_params=pltpu.CompilerParams(
            use_tc_tiling_on_sc=True, needs_layout_passes=False),
    )
    return fn(xt_flat, corrw, perm)


@jax.jit
def _emix_noise(inpute, perm):
    xt = inpute.T
    corrw = _corrections(xt)
    out_flat = _sc_combine(jnp.reshape(xt, (-1,)), corrw, perm)
    return jnp.reshape(out_flat, (N, ROWS)).T


def kernel(inpute):
    return _emix_noise(inpute, jnp.asarray(_PERM))


# speedup vs baseline: 2.1878x; 2.1878x over previous
import functools

import jax
import jax.numpy as jnp
import numpy as np
from jax import lax
from jax.experimental import pallas as pl
from jax.experimental.pallas import tpu as pltpu
from jax.experimental.pallas import tpu_sc as plsc

ROWS = 128
N = 100000
POWER = np.float32(0.1)

NC, NS = 2, 16
NW = NC * NS
CR = 200
CHUNKS = N // CR
CWORDS = CR * ROWS
FULLW = 20
NT_HI = 16
NT_LO = 15

_PERM = np.array([
    121, 35, 45, 99, 31, 112, 85, 63, 117, 114, 82, 65, 7, 4, 101, 102,
    78, 29, 108, 83, 44, 16, 58, 123, 37, 111, 19, 61, 2, 34, 5, 90,
    110, 72, 30, 42, 3, 70, 67, 39, 56, 69, 80, 22, 6, 118, 54, 77,
    18, 10, 11, 53, 94, 32, 15, 49, 50, 20, 43, 92, 8, 24, 81, 96,
    106, 9, 40, 71, 93, 59, 75, 97, 66, 25, 73, 13, 52, 88, 62, 87,
    76, 60, 47, 33, 79, 14, 17, 38, 86, 23, 105, 0, 41, 64, 21, 124,
    116, 26, 57, 89, 126, 125, 1, 115, 28, 113, 48, 36, 119, 120, 122, 100,
    91, 55, 103, 51, 127, 98, 107, 27, 74, 12, 109, 84, 68, 104, 95, 46,
], dtype=np.int32)
_PMATT = np.zeros((ROWS, ROWS), dtype=np.float32)
_PMATT[_PERM, np.arange(ROWS)] = 1.0


_RB = 5000
_RG = N // _RB


def _corr_body(xt_ref, pmat_ref, out_ref, acc_ref):
    j = pl.program_id(0)

    @pl.when(j == 0)
    def _():
        acc_ref[...] = jnp.zeros_like(acc_ref)

    acc_ref[...] += jnp.sum(xt_ref[...], axis=0, keepdims=True)

    @pl.when(j == _RG - 1)
    def _():
        permuted = jnp.dot(acc_ref[...], pmat_ref[...],
                           preferred_element_type=jnp.float32)
        out_ref[...] = jnp.broadcast_to(permuted * np.float32(POWER / N),
                                        (8, ROWS))


def _corrections(xt):
    return pl.pallas_call(
        _corr_body,
        grid=(_RG,),
        in_specs=[pl.BlockSpec((_RB, ROWS), lambda j: (j, 0)),
                  pl.BlockSpec((ROWS, ROWS), lambda j: (0, 0))],
        out_specs=pl.BlockSpec((8, ROWS), lambda j: (0, 0)),
        out_shape=jax.ShapeDtypeStruct((8, ROWS), jnp.float32),
        scratch_shapes=[pltpu.VMEM((1, ROWS), jnp.float32)],
    )(xt, jnp.asarray(_PMATT))


def _sc_body(xt_hbm, corr_hbm, perm_hbm, out_hbm,
             permv, corr_v, blk0, blk1, ob0, ob1, si0, si1, so0, so1):
    c = lax.axis_index("c")
    s = lax.axis_index("s")
    w = s * NC + c
    pltpu.sync_copy(perm_hbm, permv)
    pltpu.sync_copy(corr_hbm, corr_v)

    pidx = [permv[pl.ds(16 * g, 16)] for g in range(8)]
    cks = [corr_v[0, pl.ds(16 * g, 16)] for g in range(8)]

    blk = (blk0, blk1)
    ob = (ob0, ob1)
    sis = (si0, si1)
    sos = (so0, so1)
    nt = jnp.where(w < FULLW, NT_HI, NT_LO)

    def _src(t):
        return xt_hbm.at[pl.ds((w + NW * t) * CWORDS, CWORDS)]

    def _dst(t):
        return out_hbm.at[pl.ds((w + NW * t) * CWORDS, CWORDS)]

    def _chunk(t, b):
        pltpu.make_async_copy(_src(t), blk[b], sis[b]).wait()

        @pl.when(t >= 1)
        def _():
            pltpu.make_async_copy(ob[1 - b], _dst(t - 1), sos[1 - b]).wait()

        @pl.when(t <= nt - 2)
        def _():
            pltpu.async_copy(_src(t + 1), blk[1 - b], sis[1 - b])

        def _cmb(r, _, _b=b):
            base = r * ROWS
            nzs = [plsc.load_gather(blk[_b], [base + pidx[g]])
                   for g in range(8)]
            xs = [blk[_b][pl.ds(base + 16 * g, 16)] for g in range(8)]
            res = [xs[g] + POWER * nzs[g] - cks[g] for g in range(8)]
            for g in range(8):
                ob[_b][pl.ds(base + 16 * g, 16)] = res[g]
            return 0

        lax.fori_loop(0, CR, _cmb, 0)
        pltpu.async_copy(ob[b], _dst(t), sos[b])

    def _body(t, _):
        pl.when(t % 2 == 0)(lambda: _chunk(t, 0))
        pl.when(t % 2 == 1)(lambda: _chunk(t, 1))
        return 0

    pltpu.async_copy(_src(0), blk[0], sis[0])
    lax.fori_loop(0, nt, _body, 0)

    @pl.when(w < FULLW)
    def _():
        pltpu.make_async_copy(ob[(NT_HI - 1) % 2], _dst(NT_HI - 1),
                              sos[(NT_HI - 1) % 2]).wait()

    @pl.when(w >= FULLW)
    def _():
        pltpu.make_async_copy(ob[(NT_LO - 1) % 2], _dst(NT_LO - 1),
                              sos[(NT_LO - 1) % 2]).wait()


def _sc_combine(xt_flat, corrw, perm):
    fn = pl.kernel(
        _sc_body,
        out_type=jax.ShapeDtypeStruct((N * ROWS,), jnp.float32),
        mesh=plsc.VectorSubcoreMesh(core_axis_name="c", subcore_axis_name="s"),
        scratch_types=[
            pltpu.VMEM((ROWS,), jnp.int32),
            pltpu.VMEM((8, ROWS), jnp.float32),
            pltpu.VMEM((CWORDS,), jnp.float32),
            pltpu.VMEM((CWORDS,), jnp.float32),
            pltpu.VMEM((CWORDS,), jnp.float32),
            pltpu.VMEM((CWORDS,), jnp.float32),
            pltpu.SemaphoreType.DMA,
            pltpu.SemaphoreType.DMA,
            pltpu.SemaphoreType.DMA,
            pltpu.SemaphoreType.DMA,
        ],
        compiler_params=pltpu.CompilerParams(
            use_tc_tiling_on_sc=True, needs_layout_passes=False),
    )
    return fn(xt_flat, corrw, perm)


@jax.jit
def _emix_noise(inpute, perm):
    xt = inpute.T
    corrw = _corrections(xt)
    out_flat = _sc_combine(jnp.reshape(xt, (-1,)), corrw, perm)
    return jnp.reshape(out_flat, (N, ROWS)).T


def kernel(inpute):
    return _emix_noise(inpute, jnp.asarray(_PERM))


# speedup vs baseline: 2.3263x; 1.0633x over previous
import functools

import jax
import jax.numpy as jnp
import numpy as np
from jax import lax
from jax.experimental import pallas as pl
from jax.experimental.pallas import tpu as pltpu
from jax.experimental.pallas import tpu_sc as plsc

ROWS = 128
N = 100000
POWER = np.float32(0.1)

NC, NS = 2, 16
NW = NC * NS
CR = 200
CHUNKS = N // CR
CWORDS = CR * ROWS
FULLW = 20
NT_HI = 16
NT_LO = 15

_PERM = np.array([
    121, 35, 45, 99, 31, 112, 85, 63, 117, 114, 82, 65, 7, 4, 101, 102,
    78, 29, 108, 83, 44, 16, 58, 123, 37, 111, 19, 61, 2, 34, 5, 90,
    110, 72, 30, 42, 3, 70, 67, 39, 56, 69, 80, 22, 6, 118, 54, 77,
    18, 10, 11, 53, 94, 32, 15, 49, 50, 20, 43, 92, 8, 24, 81, 96,
    106, 9, 40, 71, 93, 59, 75, 97, 66, 25, 73, 13, 52, 88, 62, 87,
    76, 60, 47, 33, 79, 14, 17, 38, 86, 23, 105, 0, 41, 64, 21, 124,
    116, 26, 57, 89, 126, 125, 1, 115, 28, 113, 48, 36, 119, 120, 122, 100,
    91, 55, 103, 51, 127, 98, 107, 27, 74, 12, 109, 84, 68, 104, 95, 46,
], dtype=np.int32)
_PMATT = np.zeros((ROWS, ROWS), dtype=np.float32)
_PMATT[_PERM, np.arange(ROWS)] = 1.0


_RB = 5000
_RG = N // _RB


def _corr_body(xt_ref, pmat_ref, out_ref, acc_ref):
    j = pl.program_id(0)

    @pl.when(j == 0)
    def _():
        acc_ref[...] = jnp.zeros_like(acc_ref)

    acc_ref[...] += jnp.sum(xt_ref[...], axis=0, keepdims=True)

    @pl.when(j == _RG - 1)
    def _():
        permuted = jnp.dot(acc_ref[...], pmat_ref[...],
                           preferred_element_type=jnp.float32)
        out_ref[...] = jnp.broadcast_to(permuted * np.float32(POWER / N),
                                        (8, ROWS))


def _corrections(xt):
    return pl.pallas_call(
        _corr_body,
        grid=(_RG,),
        in_specs=[pl.BlockSpec((_RB, ROWS), lambda j: (j, 0)),
                  pl.BlockSpec((ROWS, ROWS), lambda j: (0, 0))],
        out_specs=pl.BlockSpec((8, ROWS), lambda j: (0, 0)),
        out_shape=jax.ShapeDtypeStruct((8, ROWS), jnp.float32),
        scratch_shapes=[pltpu.VMEM((1, ROWS), jnp.float32)],
    )(xt, jnp.asarray(_PMATT))


def _sc_body(xt_hbm, corr_hbm, perm_hbm, out_hbm,
             permv, corr_v, blk0, blk1, ob0, ob1, si0, si1, so0, so1):
    c = lax.axis_index("c")
    s = lax.axis_index("s")
    w = s * NC + c
    pltpu.sync_copy(perm_hbm, permv)
    pltpu.sync_copy(corr_hbm, corr_v)

    pidx = [permv[pl.ds(16 * g, 16)] for g in range(8)]
    cks = [corr_v[0, pl.ds(16 * g, 16)] for g in range(8)]

    blk = (blk0, blk1)
    ob = (ob0, ob1)
    sis = (si0, si1)
    sos = (so0, so1)
    nt = jnp.where(w < FULLW, NT_HI, NT_LO)

    def _src(t):
        return xt_hbm.at[pl.ds((w + NW * t) * CWORDS, CWORDS)]

    def _dst(t):
        return out_hbm.at[pl.ds((w + NW * t) * CWORDS, CWORDS)]

    def _chunk(t, b):
        pltpu.make_async_copy(_src(t), blk[b], sis[b]).wait()

        @pl.when(t >= 1)
        def _():
            pltpu.make_async_copy(ob[1 - b], _dst(t - 1), sos[1 - b]).wait()

        @pl.when(t <= nt - 2)
        def _():
            pltpu.async_copy(_src(t + 1), blk[1 - b], sis[1 - b])

        def _cmb(r, _, _b=b):
            base = r * (2 * ROWS)
            offs = [base + (g >> 3) * ROWS + (g & 7) * 16 for g in range(16)]
            nzs = [plsc.load_gather(blk[_b],
                                    [base + (g >> 3) * ROWS + pidx[g & 7]])
                   for g in range(16)]
            xs = [blk[_b][pl.ds(offs[g], 16)] for g in range(16)]
            res = [xs[g] + POWER * nzs[g] - cks[g & 7] for g in range(16)]
            for g in range(16):
                ob[_b][pl.ds(offs[g], 16)] = res[g]
            return 0

        lax.fori_loop(0, CR // 2, _cmb, 0)
        pltpu.async_copy(ob[b], _dst(t), sos[b])

    def _body(t, _):
        pl.when(t % 2 == 0)(lambda: _chunk(t, 0))
        pl.when(t % 2 == 1)(lambda: _chunk(t, 1))
        return 0

    pltpu.async_copy(_src(0), blk[0], sis[0])
    lax.fori_loop(0, nt, _body, 0)

    @pl.when(w < FULLW)
    def _():
        pltpu.make_async_copy(ob[(NT_HI - 1) % 2], _dst(NT_HI - 1),
                              sos[(NT_HI - 1) % 2]).wait()

    @pl.when(w >= FULLW)
    def _():
        pltpu.make_async_copy(ob[(NT_LO - 1) % 2], _dst(NT_LO - 1),
                              sos[(NT_LO - 1) % 2]).wait()


def _sc_combine(xt_flat, corrw, perm):
    fn = pl.kernel(
        _sc_body,
        out_type=jax.ShapeDtypeStruct((N * ROWS,), jnp.float32),
        mesh=plsc.VectorSubcoreMesh(core_axis_name="c", subcore_axis_name="s"),
        scratch_types=[
            pltpu.VMEM((ROWS,), jnp.int32),
            pltpu.VMEM((8, ROWS), jnp.float32),
            pltpu.VMEM((CWORDS,), jnp.float32),
            pltpu.VMEM((CWORDS,), jnp.float32),
            pltpu.VMEM((CWORDS,), jnp.float32),
            pltpu.VMEM((CWORDS,), jnp.float32),
            pltpu.SemaphoreType.DMA,
            pltpu.SemaphoreType.DMA,
            pltpu.SemaphoreType.DMA,
            pltpu.SemaphoreType.DMA,
        ],
        compiler_params=pltpu.CompilerParams(
            use_tc_tiling_on_sc=True, needs_layout_passes=False),
    )
    return fn(xt_flat, corrw, perm)


@jax.jit
def _emix_noise(inpute, perm):
    xt = inpute.T
    corrw = _corrections(xt)
    out_flat = _sc_combine(jnp.reshape(xt, (-1,)), corrw, perm)
    return jnp.reshape(out_flat, (N, ROWS)).T


def kernel(inpute):
    return _emix_noise(inpute, jnp.asarray(_PERM))


# speedup vs baseline: 2.4301x; 1.0446x over previous
import functools

import jax
import jax.numpy as jnp
import numpy as np
from jax import lax
from jax.experimental import pallas as pl
from jax.experimental.pallas import tpu as pltpu
from jax.experimental.pallas import tpu_sc as plsc

ROWS = 128
N = 100000
POWER = np.float32(0.1)

NC, NS = 2, 16
NW = NC * NS
CR = 200
CHUNKS = N // CR
CWORDS = CR * ROWS
FULLW = 20
NT_HI = 16
NT_LO = 15

_PERM = np.array([
    121, 35, 45, 99, 31, 112, 85, 63, 117, 114, 82, 65, 7, 4, 101, 102,
    78, 29, 108, 83, 44, 16, 58, 123, 37, 111, 19, 61, 2, 34, 5, 90,
    110, 72, 30, 42, 3, 70, 67, 39, 56, 69, 80, 22, 6, 118, 54, 77,
    18, 10, 11, 53, 94, 32, 15, 49, 50, 20, 43, 92, 8, 24, 81, 96,
    106, 9, 40, 71, 93, 59, 75, 97, 66, 25, 73, 13, 52, 88, 62, 87,
    76, 60, 47, 33, 79, 14, 17, 38, 86, 23, 105, 0, 41, 64, 21, 124,
    116, 26, 57, 89, 126, 125, 1, 115, 28, 113, 48, 36, 119, 120, 122, 100,
    91, 55, 103, 51, 127, 98, 107, 27, 74, 12, 109, 84, 68, 104, 95, 46,
], dtype=np.int32)
_PMATT = np.zeros((ROWS, ROWS), dtype=np.float32)
_PMATT[_PERM, np.arange(ROWS)] = 1.0


_RB = 10000
_RG = N // _RB


def _corr_body(xt_ref, pmat_ref, out_ref, acc_ref):
    j = pl.program_id(0)

    @pl.when(j == 0)
    def _():
        acc_ref[...] = jnp.zeros_like(acc_ref)

    acc_ref[...] += jnp.sum(xt_ref[...], axis=0, keepdims=True)

    @pl.when(j == _RG - 1)
    def _():
        permuted = jnp.dot(acc_ref[...], pmat_ref[...],
                           preferred_element_type=jnp.float32)
        out_ref[...] = jnp.broadcast_to(permuted * np.float32(POWER / N),
                                        (8, ROWS))


def _corrections(xt):
    return pl.pallas_call(
        _corr_body,
        grid=(_RG,),
        in_specs=[pl.BlockSpec((_RB, ROWS), lambda j: (j, 0)),
                  pl.BlockSpec((ROWS, ROWS), lambda j: (0, 0))],
        out_specs=pl.BlockSpec((8, ROWS), lambda j: (0, 0)),
        out_shape=jax.ShapeDtypeStruct((8, ROWS), jnp.float32),
        scratch_shapes=[pltpu.VMEM((1, ROWS), jnp.float32)],
    )(xt, jnp.asarray(_PMATT))


def _sc_body(xt_hbm, corr_hbm, perm_hbm, out_hbm,
             permv, corr_v, blk0, blk1, ob0, ob1, si0, si1, so0, so1):
    c = lax.axis_index("c")
    s = lax.axis_index("s")
    w = s * NC + c
    pltpu.sync_copy(perm_hbm, permv)
    pltpu.sync_copy(corr_hbm, corr_v)

    pidx = [permv[pl.ds(16 * g, 16)] for g in range(8)]
    cks = [corr_v[0, pl.ds(16 * g, 16)] for g in range(8)]

    blk = (blk0, blk1)
    ob = (ob0, ob1)
    sis = (si0, si1)
    sos = (so0, so1)
    nt = jnp.where(w < FULLW, NT_HI, NT_LO)

    def _src(t):
        return xt_hbm.at[pl.ds((w + NW * t) * CWORDS, CWORDS)]

    def _dst(t):
        return out_hbm.at[pl.ds((w + NW * t) * CWORDS, CWORDS)]

    def _chunk(t, b):
        pltpu.make_async_copy(_src(t), blk[b], sis[b]).wait()

        @pl.when(t >= 1)
        def _():
            pltpu.make_async_copy(ob[1 - b], _dst(t - 1), sos[1 - b]).wait()

        @pl.when(t <= nt - 2)
        def _():
            pltpu.async_copy(_src(t + 1), blk[1 - b], sis[1 - b])

        def _cmb(r, _, _b=b):
            base = r * (2 * ROWS)
            offs = [base + (g >> 3) * ROWS + (g & 7) * 16 for g in range(16)]
            nzs = [plsc.load_gather(blk[_b],
                                    [base + (g >> 3) * ROWS + pidx[g & 7]])
                   for g in range(16)]
            xs = [blk[_b][pl.ds(offs[g], 16)] for g in range(16)]
            res = [xs[g] + POWER * nzs[g] - cks[g & 7] for g in range(16)]
            for g in range(16):
                ob[_b][pl.ds(offs[g], 16)] = res[g]
            return 0

        lax.fori_loop(0, CR // 2, _cmb, 0)
        pltpu.async_copy(ob[b], _dst(t), sos[b])

    def _body(t, _):
        pl.when(t % 2 == 0)(lambda: _chunk(t, 0))
        pl.when(t % 2 == 1)(lambda: _chunk(t, 1))
        return 0

    pltpu.async_copy(_src(0), blk[0], sis[0])
    lax.fori_loop(0, nt, _body, 0)

    @pl.when(w < FULLW)
    def _():
        pltpu.make_async_copy(ob[(NT_HI - 1) % 2], _dst(NT_HI - 1),
                              sos[(NT_HI - 1) % 2]).wait()

    @pl.when(w >= FULLW)
    def _():
        pltpu.make_async_copy(ob[(NT_LO - 1) % 2], _dst(NT_LO - 1),
                              sos[(NT_LO - 1) % 2]).wait()


def _sc_combine(xt_flat, corrw, perm):
    fn = pl.kernel(
        _sc_body,
        out_type=jax.ShapeDtypeStruct((N * ROWS,), jnp.float32),
        mesh=plsc.VectorSubcoreMesh(core_axis_name="c", subcore_axis_name="s"),
        scratch_types=[
            pltpu.VMEM((ROWS,), jnp.int32),
            pltpu.VMEM((8, ROWS), jnp.float32),
            pltpu.VMEM((CWORDS,), jnp.float32),
            pltpu.VMEM((CWORDS,), jnp.float32),
            pltpu.VMEM((CWORDS,), jnp.float32),
            pltpu.VMEM((CWORDS,), jnp.float32),
            pltpu.SemaphoreType.DMA,
            pltpu.SemaphoreType.DMA,
            pltpu.SemaphoreType.DMA,
            pltpu.SemaphoreType.DMA,
        ],
        compiler_params=pltpu.CompilerParams(
            use_tc_tiling_on_sc=True, needs_layout_passes=False),
    )
    return fn(xt_flat, corrw, perm)


@jax.jit
def _emix_noise(inpute, perm):
    xt = inpute.T
    corrw = _corrections(xt)
    out_flat = _sc_combine(jnp.reshape(xt, (-1,)), corrw, perm)
    return jnp.reshape(out_flat, (N, ROWS)).T


def kernel(inpute):
    return _emix_noise(inpute, jnp.asarray(_PERM))


# speedup vs baseline: 2.4334x; 1.0014x over previous
import jax
import jax.numpy as jnp
import numpy as np
from jax import lax
from jax.experimental import pallas as pl
from jax.experimental.pallas import tpu as pltpu
from jax.experimental.pallas import tpu_sc as plsc

ROWS = 128
N = 100000
POWER = np.float32(0.1)

NC, NS = 2, 16
NW = NC * NS
CR = 200
CHUNKS = N // CR
CWORDS = CR * ROWS
FULLW = 20
NT_HI = 16
NT_LO = 15

_PERM = np.array([
    121, 35, 45, 99, 31, 112, 85, 63, 117, 114, 82, 65, 7, 4, 101, 102,
    78, 29, 108, 83, 44, 16, 58, 123, 37, 111, 19, 61, 2, 34, 5, 90,
    110, 72, 30, 42, 3, 70, 67, 39, 56, 69, 80, 22, 6, 118, 54, 77,
    18, 10, 11, 53, 94, 32, 15, 49, 50, 20, 43, 92, 8, 24, 81, 96,
    106, 9, 40, 71, 93, 59, 75, 97, 66, 25, 73, 13, 52, 88, 62, 87,
    76, 60, 47, 33, 79, 14, 17, 38, 86, 23, 105, 0, 41, 64, 21, 124,
    116, 26, 57, 89, 126, 125, 1, 115, 28, 113, 48, 36, 119, 120, 122, 100,
    91, 55, 103, 51, 127, 98, 107, 27, 74, 12, 109, 84, 68, 104, 95, 46,
], dtype=np.int32)
_PMATT = np.zeros((ROWS, ROWS), dtype=np.float32)
_PMATT[_PERM, np.arange(ROWS)] = 1.0


_RB = 10000
_RG = N // _RB


def _corr_body(xt_ref, pmat_ref, out_ref, acc_ref):
    j = pl.program_id(0)

    @pl.when(j == 0)
    def _():
        acc_ref[...] = jnp.zeros_like(acc_ref)

    acc_ref[...] += jnp.sum(xt_ref[...], axis=0, keepdims=True)

    @pl.when(j == _RG - 1)
    def _():
        permuted = jnp.dot(acc_ref[...], pmat_ref[...],
                           preferred_element_type=jnp.float32)
        out_ref[...] = jnp.broadcast_to(permuted * np.float32(POWER / N),
                                        (8, ROWS))


def _corrections(xt):
    return pl.pallas_call(
        _corr_body,
        grid=(_RG,),
        in_specs=[pl.BlockSpec((_RB, ROWS), lambda j: (j, 0)),
                  pl.BlockSpec((ROWS, ROWS), lambda j: (0, 0))],
        out_specs=pl.BlockSpec((8, ROWS), lambda j: (0, 0)),
        out_shape=jax.ShapeDtypeStruct((8, ROWS), jnp.float32),
        scratch_shapes=[pltpu.VMEM((1, ROWS), jnp.float32)],
    )(xt, jnp.asarray(_PMATT))


def _sc_body(xt_hbm, corr_hbm, perm_hbm, out_hbm,
             permv, corr_v, blk0, blk1, ob0, ob1, si0, si1, so0, so1):
    c = lax.axis_index("c")
    s = lax.axis_index("s")
    w = s * NC + c
    pltpu.sync_copy(perm_hbm, permv)
    pltpu.sync_copy(corr_hbm, corr_v)

    pidx = [permv[pl.ds(16 * g, 16)] for g in range(8)]
    cks = [corr_v[0, pl.ds(16 * g, 16)] for g in range(8)]

    blk = (blk0, blk1)
    ob = (ob0, ob1)
    sis = (si0, si1)
    sos = (so0, so1)
    nt = jnp.where(w < FULLW, NT_HI, NT_LO)

    def _src(t):
        return xt_hbm.at[pl.ds((w + NW * t) * CWORDS, CWORDS)]

    def _dst(t):
        return out_hbm.at[pl.ds((w + NW * t) * CWORDS, CWORDS)]

    def _chunk(t, b):
        pltpu.make_async_copy(_src(t), blk[b], sis[b]).wait()

        @pl.when(t >= 1)
        def _():
            pltpu.make_async_copy(ob[1 - b], _dst(t - 1), sos[1 - b]).wait()

        @pl.when(t <= nt - 2)
        def _():
            pltpu.async_copy(_src(t + 1), blk[1 - b], sis[1 - b])

        def _cmb(r, _, _b=b):
            base = r * (2 * ROWS)
            offs = [base + (g >> 3) * ROWS + (g & 7) * 16 for g in range(16)]
            nzs = [plsc.load_gather(blk[_b],
                                    [base + (g >> 3) * ROWS + pidx[g & 7]])
                   for g in range(16)]
            xs = [blk[_b][pl.ds(offs[g], 16)] for g in range(16)]
            res = [xs[g] + POWER * nzs[g] - cks[g & 7] for g in range(16)]
            for g in range(16):
                ob[_b][pl.ds(offs[g], 16)] = res[g]
            return 0

        lax.fori_loop(0, CR // 2, _cmb, 0)
        pltpu.async_copy(ob[b], _dst(t), sos[b])

    def _body(t, _):
        pl.when(t % 2 == 0)(lambda: _chunk(t, 0))
        pl.when(t % 2 == 1)(lambda: _chunk(t, 1))
        return 0

    pltpu.async_copy(_src(0), blk[0], sis[0])
    lax.fori_loop(0, nt, _body, 0)

    @pl.when(w < FULLW)
    def _():
        pltpu.make_async_copy(ob[(NT_HI - 1) % 2], _dst(NT_HI - 1),
                              sos[(NT_HI - 1) % 2]).wait()

    @pl.when(w >= FULLW)
    def _():
        pltpu.make_async_copy(ob[(NT_LO - 1) % 2], _dst(NT_LO - 1),
                              sos[(NT_LO - 1) % 2]).wait()


def _sc_combine(xt_flat, corrw, perm):
    fn = pl.kernel(
        _sc_body,
        out_type=jax.ShapeDtypeStruct((N * ROWS,), jnp.float32),
        mesh=plsc.VectorSubcoreMesh(core_axis_name="c", subcore_axis_name="s"),
        scratch_types=[
            pltpu.VMEM((ROWS,), jnp.int32),
            pltpu.VMEM((8, ROWS), jnp.float32),
            pltpu.VMEM((CWORDS,), jnp.float32),
            pltpu.VMEM((CWORDS,), jnp.float32),
            pltpu.VMEM((CWORDS,), jnp.float32),
            pltpu.VMEM((CWORDS,), jnp.float32),
            pltpu.SemaphoreType.DMA,
            pltpu.SemaphoreType.DMA,
            pltpu.SemaphoreType.DMA,
            pltpu.SemaphoreType.DMA,
        ],
        compiler_params=pltpu.CompilerParams(
            use_tc_tiling_on_sc=True, needs_layout_passes=False),
    )
    return fn(xt_flat, corrw, perm)


@jax.jit
def _emix_noise(inpute, perm):
    xt = inpute.T
    corrw = _corrections(xt)
    out_flat = _sc_combine(jnp.reshape(xt, (-1,)), corrw, perm)
    return jnp.reshape(out_flat, (N, ROWS)).T


def kernel(inpute):
    return _emix_noise(inpute, jnp.asarray(_PERM))
